# Initial kernel scaffold; baseline (speedup 1.0000x reference)
import jax
import jax.numpy as jnp
from jax.experimental import pallas as pl


def kernel(x, Wg, W1, b1, W2, b2, Ws1, bs1, Ws2, bs2):
    raise NotImplementedError("write your pallas kernel here")


# speedup vs baseline: 9.5765x; 9.5765x over previous
import functools

import jax
import jax.numpy as jnp
from jax.experimental import pallas as pl
from jax.experimental.pallas import tpu as pltpu

D = 768
F = 3072
E = 64
K = 2
T = 128
NEG = -1e30


def _gelu(h):
    return 0.5 * h * (1.0 + jax.lax.erf(h * 0.7071067811865476))


def _gate_body(x_ref, wg_ref, idx_ref, w_ref):
    logits = jnp.dot(x_ref[...], wg_ref[...], preferred_element_type=jnp.float32)
    n = logits.shape[0]
    cols = jax.lax.broadcasted_iota(jnp.int32, logits.shape, 1)
    l0 = jnp.max(logits, axis=1)
    i0 = jnp.min(jnp.where(logits == l0[:, None], cols, E), axis=1)
    masked = jnp.where(cols == i0[:, None], NEG, logits)
    l1 = jnp.max(masked, axis=1)
    i1 = jnp.min(jnp.where(masked == l1[:, None], cols, E), axis=1)
    w0 = 1.0 / (1.0 + jnp.exp(l1 - l0))
    idx_ref[...] = jnp.stack([i0, i1], axis=1)
    w_ref[...] = jnp.stack([w0, 1.0 - w0], axis=1)


def _moe_body(texp_ref, act_ref, gidx_ref, gw_ref, x_ref,
              w1_ref, b1_ref, w2_ref, b2_ref, out_ref):
    i = pl.program_id(0)

    @pl.when(i == 0)
    def _init():
        out_ref[...] = jnp.zeros_like(out_ref)

    @pl.when(act_ref[i] == 1)
    def _work():
        n = x_ref.shape[0]
        gi = gidx_ref[0, 0, :]
        cols = jax.lax.broadcasted_iota(jnp.int32, (T, n), 1)
        oh = (gi[:, None] == cols).astype(jnp.float32)
        xg = jnp.dot(oh, x_ref[...], preferred_element_type=jnp.float32)
        h = jnp.dot(xg, w1_ref[0], preferred_element_type=jnp.float32)
        h = _gelu(h + b1_ref[0, 0, :][None, :])
        y = jnp.dot(h, w2_ref[0], preferred_element_type=jnp.float32)
        y = (y + b2_ref[0, 0, :][None, :]) * gw_ref[0, 0, :][:, None]
        out_ref[...] += jax.lax.dot_general(
            oh, y, (((0,), (0,)), ((), ())),
            preferred_element_type=jnp.float32)


def _shared_body(x_ref, moe_ref, ws1_ref, bs1_ref, ws2_ref, bs2_ref, out_ref):
    h = jnp.dot(x_ref[...], ws1_ref[...], preferred_element_type=jnp.float32)
    h = _gelu(h + bs1_ref[...])
    y = jnp.dot(h, ws2_ref[...], preferred_element_type=jnp.float32)
    out_ref[...] = y + bs2_ref[...] + moe_ref[...]


def kernel(x, Wg, W1, b1, W2, b2, Ws1, bs1, Ws2, bs2):
    Bn, Ln, Dn = x.shape
    n = Bn * Ln
    x2 = x.reshape(n, Dn)
    nt = n * K // T + E
    maxp = nt * T

    idx, w = pl.pallas_call(
        _gate_body,
        out_shape=(jax.ShapeDtypeStruct((n, K), jnp.int32),
                   jax.ShapeDtypeStruct((n, K), jnp.float32)),
    )(x2, Wg)

    flat_e = idx.reshape(-1)
    flat_w = w.reshape(-1)
    flat_t = jnp.arange(n * K, dtype=jnp.int32) // K
    onehot = (flat_e[:, None] == jnp.arange(E, dtype=jnp.int32)[None, :])
    oh_i = onehot.astype(jnp.int32)
    counts = jnp.sum(oh_i, axis=0)
    tiles_per_e = (counts + T - 1) // T
    tile_end = jnp.cumsum(tiles_per_e)
    tile_start = tile_end - tiles_per_e
    total_tiles = tile_end[-1]
    rank = jnp.sum(jnp.cumsum(oh_i, axis=0) * oh_i, axis=1) - 1
    pos = tile_start[flat_e] * T + rank
    gidx = jnp.zeros((maxp,), jnp.int32).at[pos].set(flat_t)
    gw = jnp.zeros((maxp,), jnp.float32).at[pos].set(flat_w)
    tids = jnp.arange(nt, dtype=jnp.int32)
    texp = jnp.searchsorted(tile_end, tids, side='right').astype(jnp.int32)
    active = (tids < total_tiles)
    last_e = jnp.max(jnp.where(active, texp, -1))
    texp = jnp.where(active, texp, last_e)
    act = active.astype(jnp.int32)

    gidx3 = gidx.reshape(nt, 1, T)
    gw3 = gw.reshape(nt, 1, T)
    b13 = b1.reshape(E, 1, F)
    b23 = b2.reshape(E, 1, D)

    grid_spec = pltpu.PrefetchScalarGridSpec(
        num_scalar_prefetch=2,
        grid=(nt,),
        in_specs=[
            pl.BlockSpec((1, 1, T), lambda i, texp, act: (i, 0, 0)),
            pl.BlockSpec((1, 1, T), lambda i, texp, act: (i, 0, 0)),
            pl.BlockSpec((n, D), lambda i, texp, act: (0, 0)),
            pl.BlockSpec((1, D, F), lambda i, texp, act: (texp[i], 0, 0)),
            pl.BlockSpec((1, 1, F), lambda i, texp, act: (texp[i], 0, 0)),
            pl.BlockSpec((1, F, D), lambda i, texp, act: (texp[i], 0, 0)),
            pl.BlockSpec((1, 1, D), lambda i, texp, act: (texp[i], 0, 0)),
        ],
        out_specs=pl.BlockSpec((n, D), lambda i, texp, act: (0, 0)),
    )
    moe_out = pl.pallas_call(
        _moe_body,
        grid_spec=grid_spec,
        out_shape=jax.ShapeDtypeStruct((n, D), jnp.float32),
    )(texp, act, gidx3, gw3, x2, W1, b13, W2, b23)

    tb = 128
    out = pl.pallas_call(
        _shared_body,
        grid=(n // tb,),
        in_specs=[
            pl.BlockSpec((tb, D), lambda i: (i, 0)),
            pl.BlockSpec((tb, D), lambda i: (i, 0)),
            pl.BlockSpec((D, F), lambda i: (0, 0)),
            pl.BlockSpec((1, F), lambda i: (0, 0)),
            pl.BlockSpec((F, D), lambda i: (0, 0)),
            pl.BlockSpec((1, D), lambda i: (0, 0)),
        ],
        out_specs=pl.BlockSpec((tb, D), lambda i: (i, 0)),
        out_shape=jax.ShapeDtypeStruct((n, D), jnp.float32),
    )(x2, moe_out, Ws1, bs1.reshape(1, F), Ws2, bs2.reshape(1, D))

    return out.reshape(Bn, Ln, Dn)
